# baseline (device time: 99456 ns/iter reference)
import functools

import jax
import jax.numpy as jnp
from jax import lax
from jax.experimental import pallas as pl
from jax.experimental.pallas import tpu as pltpu

N_DEV = 32
M_BLK = 128
TILE_N = 512


def _all_pairs_barrier(sem, me):
    for h in range(1, N_DEV):
        tgt = lax.rem(me + h, N_DEV)
        pl.semaphore_signal(
            sem, 1, device_id=(tgt,), device_id_type=pl.DeviceIdType.MESH
        )
    pl.semaphore_wait(sem, N_DEV - 1)


def _a2a_body(x_ref, out_ref, send_sems, recv_sems):
    me = lax.axis_index("i")
    _all_pairs_barrier(pltpu.get_barrier_semaphore(), me)

    out_ref[:, pl.ds(me * M_BLK, M_BLK)] = x_ref[pl.ds(me * M_BLK, M_BLK), :]

    sends = []
    for h in range(1, N_DEV):
        tgt = lax.rem(me + h, N_DEV)
        rdma = pltpu.make_async_remote_copy(
            src_ref=x_ref.at[pl.ds(tgt * M_BLK, M_BLK), :],
            dst_ref=out_ref.at[:, pl.ds(me * M_BLK, M_BLK)],
            send_sem=send_sems.at[tgt],
            recv_sem=recv_sems.at[me],
            device_id=(tgt,),
            device_id_type=pl.DeviceIdType.MESH,
        )
        rdma.start()
        sends.append(rdma)

    for h in range(1, N_DEV):
        src = lax.rem(me + h, N_DEV)
        recv = pltpu.make_async_remote_copy(
            src_ref=x_ref.at[pl.ds(src * M_BLK, M_BLK), :],
            dst_ref=out_ref.at[:, pl.ds(src * M_BLK, M_BLK)],
            send_sem=send_sems.at[src],
            recv_sem=recv_sems.at[src],
            device_id=(src,),
            device_id_type=pl.DeviceIdType.MESH,
        )
        recv.wait_recv()

    for rdma in sends:
        rdma.wait_send()

    @functools.partial(pl.run_scoped, exit_sem=pltpu.SemaphoreType.REGULAR)
    def _(exit_sem):
        _all_pairs_barrier(exit_sem, me)


def _gemm_body(x_ref, w_ref, y_ref, amax_ref):
    j = pl.program_id(0)
    y = jnp.dot(x_ref[...], w_ref[...], preferred_element_type=jnp.float32)
    y = jnp.maximum(y, 0.0)
    y_ref[...] = y
    m = jnp.max(y)

    @pl.when(j == 0)
    def _():
        amax_ref[...] = jnp.full((1, 128), m, jnp.float32)

    @pl.when(j != 0)
    def _():
        amax_ref[...] = jnp.maximum(amax_ref[...], m)


def _allmax_body(a_ref, out_ref, comm_ref, send_sems, recv_sems):
    me = lax.axis_index("i")
    _all_pairs_barrier(pltpu.get_barrier_semaphore(), me)

    comm_ref[pl.ds(me, 1), :] = a_ref[...]
    sends = []
    for h in range(1, N_DEV):
        tgt = lax.rem(me + h, N_DEV)
        rdma = pltpu.make_async_remote_copy(
            src_ref=a_ref,
            dst_ref=comm_ref.at[pl.ds(me, 1), :],
            send_sem=send_sems.at[tgt],
            recv_sem=recv_sems.at[me],
            device_id=(tgt,),
            device_id_type=pl.DeviceIdType.MESH,
        )
        rdma.start()
        sends.append(rdma)

    for h in range(1, N_DEV):
        src = lax.rem(me + h, N_DEV)
        recv = pltpu.make_async_remote_copy(
            src_ref=a_ref,
            dst_ref=comm_ref.at[pl.ds(src, 1), :],
            send_sem=send_sems.at[src],
            recv_sem=recv_sems.at[src],
            device_id=(src,),
            device_id_type=pl.DeviceIdType.MESH,
        )
        recv.wait_recv()

    out_ref[...] = jnp.full((1, 128), jnp.max(comm_ref[...]), jnp.float32)

    for rdma in sends:
        rdma.wait_send()

    @functools.partial(pl.run_scoped, exit_sem=pltpu.SemaphoreType.REGULAR)
    def _(exit_sem):
        _all_pairs_barrier(exit_sem, me)


def _quant_body(y_ref, amax_ref, out_ref):
    scale = amax_ref[0, 0] / 127.0
    q = jnp.clip(jnp.round(y_ref[...] / scale), -127.0, 127.0)
    out_ref[...] = q * scale


def kernel(x, w_mat):
    k, _ = x.shape
    _, n = w_mat.shape

    x_rows = pl.pallas_call(
        _a2a_body,
        out_shape=jax.ShapeDtypeStruct((M_BLK, k), jnp.float32),
        in_specs=[pl.BlockSpec(memory_space=pltpu.VMEM)],
        out_specs=pl.BlockSpec(memory_space=pltpu.VMEM),
        scratch_shapes=[
            pltpu.SemaphoreType.DMA((N_DEV,)),
            pltpu.SemaphoreType.DMA((N_DEV,)),
        ],
        compiler_params=pltpu.CompilerParams(collective_id=0),
    )(x)

    n_tiles = n // TILE_N
    y, amax_loc = pl.pallas_call(
        _gemm_body,
        grid=(n_tiles,),
        out_shape=[
            jax.ShapeDtypeStruct((M_BLK, n), jnp.float32),
            jax.ShapeDtypeStruct((1, 128), jnp.float32),
        ],
        in_specs=[
            pl.BlockSpec((M_BLK, k), lambda j: (0, 0)),
            pl.BlockSpec((k, TILE_N), lambda j: (0, j)),
        ],
        out_specs=[
            pl.BlockSpec((M_BLK, TILE_N), lambda j: (0, j)),
            pl.BlockSpec((1, 128), lambda j: (0, 0)),
        ],
    )(x_rows, w_mat)

    amax = pl.pallas_call(
        _allmax_body,
        out_shape=jax.ShapeDtypeStruct((1, 128), jnp.float32),
        in_specs=[pl.BlockSpec(memory_space=pltpu.VMEM)],
        out_specs=pl.BlockSpec(memory_space=pltpu.VMEM),
        scratch_shapes=[
            pltpu.VMEM((N_DEV, 128), jnp.float32),
            pltpu.SemaphoreType.DMA((N_DEV,)),
            pltpu.SemaphoreType.DMA((N_DEV,)),
        ],
        compiler_params=pltpu.CompilerParams(collective_id=1),
    )(amax_loc)

    out = pl.pallas_call(
        _quant_body,
        out_shape=jax.ShapeDtypeStruct((M_BLK, n), jnp.float32),
        in_specs=[
            pl.BlockSpec(memory_space=pltpu.VMEM),
            pl.BlockSpec(memory_space=pltpu.VMEM),
        ],
        out_specs=pl.BlockSpec(memory_space=pltpu.VMEM),
    )(y, amax)
    return out


# device time: 66530 ns/iter; 1.4949x vs baseline; 1.4949x over previous
import jax
import jax.numpy as jnp
from jax import lax
from jax.experimental import pallas as pl
from jax.experimental.pallas import tpu as pltpu

N_DEV = 32
M_BLK = 128
N_BUF = 4


def _body(x_ref, w_ref, out_ref, xg, wbuf, avec, acomm,
          wsems, send_sems, recv_sems, asend_sems, arecv_sems):
    me = lax.axis_index("i")
    k_tot = x_ref.shape[0]
    n = out_ref.shape[1]

    def issue_w(s):
        j = lax.rem(me + s, N_DEV)
        cp = pltpu.make_async_copy(
            w_ref.at[pl.ds(j * M_BLK, M_BLK), :],
            wbuf.at[s % N_BUF],
            wsems.at[s % N_BUF],
        )
        cp.start()
        return cp

    wdma = {s: issue_w(s) for s in range(N_BUF)}

    barrier = pltpu.get_barrier_semaphore()
    for h in range(1, N_DEV):
        tgt = lax.rem(me + h, N_DEV)
        pl.semaphore_signal(
            barrier, 1, device_id=(tgt,), device_id_type=pl.DeviceIdType.MESH
        )
    pl.semaphore_wait(barrier, N_DEV - 1)

    sends = []
    for s in range(1, N_DEV):
        tgt = lax.rem(me - s + N_DEV, N_DEV)
        rdma = pltpu.make_async_remote_copy(
            src_ref=x_ref.at[pl.ds(tgt * M_BLK, M_BLK), :],
            dst_ref=xg.at[:, pl.ds(s * M_BLK, M_BLK)],
            send_sem=send_sems.at[s],
            recv_sem=recv_sems.at[s],
            device_id=(tgt,),
            device_id_type=pl.DeviceIdType.MESH,
        )
        rdma.start()
        sends.append(rdma)

    for s in range(N_DEV):
        wdma[s].wait()
        if s == 0:
            xb = x_ref[pl.ds(me * M_BLK, M_BLK), :]
        else:
            recv = pltpu.make_async_remote_copy(
                src_ref=x_ref.at[pl.ds(0, M_BLK), :],
                dst_ref=xg.at[:, pl.ds(s * M_BLK, M_BLK)],
                send_sem=send_sems.at[s],
                recv_sem=recv_sems.at[s],
                device_id=(me,),
                device_id_type=pl.DeviceIdType.MESH,
            )
            recv.wait_recv()
            xb = xg[:, pl.ds(s * M_BLK, M_BLK)]
        contrib = jnp.dot(
            xb, wbuf[s % N_BUF], preferred_element_type=jnp.float32
        )
        if s == 0:
            out_ref[...] = contrib
        else:
            out_ref[...] += contrib
        if s + N_BUF < N_DEV:
            wdma[s + N_BUF] = issue_w(s + N_BUF)

    y = jnp.maximum(out_ref[...], 0.0)
    out_ref[...] = y
    avec[...] = jnp.full((1, 128), jnp.max(y), jnp.float32)

    asends = []
    for h in range(1, N_DEV):
        tgt = lax.rem(me + h, N_DEV)
        rdma = pltpu.make_async_remote_copy(
            src_ref=avec,
            dst_ref=acomm.at[pl.ds(me, 1), :],
            send_sem=asend_sems.at[tgt],
            recv_sem=arecv_sems.at[me],
            device_id=(tgt,),
            device_id_type=pl.DeviceIdType.MESH,
        )
        rdma.start()
        asends.append(rdma)
    acomm[pl.ds(me, 1), :] = avec[...]

    for rdma in sends:
        rdma.wait_send()

    for h in range(1, N_DEV):
        src = lax.rem(me + h, N_DEV)
        recv = pltpu.make_async_remote_copy(
            src_ref=avec,
            dst_ref=acomm.at[pl.ds(src, 1), :],
            send_sem=asend_sems.at[src],
            recv_sem=arecv_sems.at[src],
            device_id=(src,),
            device_id_type=pl.DeviceIdType.MESH,
        )
        recv.wait_recv()

    scale = jnp.max(acomm[...]) / 127.0
    q = jnp.clip(jnp.round(out_ref[...] / scale), -127.0, 127.0)
    out_ref[...] = q * scale

    for rdma in asends:
        rdma.wait_send()


def kernel(x, w_mat):
    k, _ = x.shape
    _, n = w_mat.shape

    return pl.pallas_call(
        _body,
        out_shape=jax.ShapeDtypeStruct((M_BLK, n), jnp.float32),
        in_specs=[
            pl.BlockSpec(memory_space=pltpu.VMEM),
            pl.BlockSpec(memory_space=pltpu.MemorySpace.HBM),
        ],
        out_specs=pl.BlockSpec(memory_space=pltpu.VMEM),
        scratch_shapes=[
            pltpu.VMEM((M_BLK, k), jnp.float32),
            pltpu.VMEM((N_BUF, M_BLK, n), jnp.float32),
            pltpu.VMEM((1, 128), jnp.float32),
            pltpu.VMEM((N_DEV, 128), jnp.float32),
            pltpu.SemaphoreType.DMA((N_BUF,)),
            pltpu.SemaphoreType.DMA((N_DEV,)),
            pltpu.SemaphoreType.DMA((N_DEV,)),
            pltpu.SemaphoreType.DMA((N_DEV,)),
            pltpu.SemaphoreType.DMA((N_DEV,)),
        ],
        compiler_params=pltpu.CompilerParams(collective_id=0),
    )(x, w_mat)
